# Initial kernel scaffold; baseline (speedup 1.0000x reference)
import jax
import jax.numpy as jnp
from jax.experimental import pallas as pl


def kernel(x, W_in, b_in, A_log, B, C, D, dt, W_out, b_out):
    raise NotImplementedError("write your pallas kernel here")


# speedup vs baseline: 3.2379x; 3.2379x over previous
import jax
import jax.numpy as jnp
from jax import lax
from jax.experimental import pallas as pl
from jax.experimental.pallas import tpu as pltpu

B_ = 4
L_ = 2048
D_ = 512
N_ = 64
T_ = 128
C_ = L_ // T_
BC_ = B_ * C_
DT1 = 256
DTC = 8
DTO = 256

_F32 = jnp.float32
_HI = lax.Precision.HIGHEST


def _inproj_kernel(x_ref, w_ref, b_ref, o_ref):
    xt = x_ref[0]
    w = w_ref[...]
    acc = lax.dot_general(w, xt, (((1,), (1,)), ((), ())),
                          preferred_element_type=_F32)
    o_ref[...] = acc + b_ref[...]


def _conv_kernel(u_ref, a_ref, bb_ref, cc_ref, dt_ref, dd_ref, o_ref):
    tau = lax.broadcasted_iota(jnp.int32, (T_, N_), 0).astype(_F32)
    row_i = lax.broadcasted_iota(jnp.int32, (T_, T_), 0)
    col_i = lax.broadcasted_iota(jnp.int32, (T_, T_), 1)
    causal = col_i >= row_i
    gi = lax.broadcasted_iota(jnp.int32, (BC_, N_), 0) % C_

    for i in range(DTC):
        a_row = a_ref[i:i + 1, :]
        dt_d = dt_ref[i:i + 1, :]
        logr = -jnp.exp(a_row) * dt_d
        coef = bb_ref[i:i + 1, :] * cc_ref[i:i + 1, :] * dt_d
        pows = jnp.exp(tau * logr)
        r1 = jnp.exp(logr)
        rT = jnp.exp(float(T_) * logr)
        rTm1 = rT * (1.0 / r1)
        rneg = 1.0 / pows
        revp = rTm1 * rneg
        cp = coef * pows
        wop = cp * r1
        u_d = u_ref[i]

        pt = lax.dot_general(rneg, cp, (((1,), (1,)), ((), ())),
                             preferred_element_type=_F32, precision=_HI)
        pt = jnp.where(causal, pt, 0.0)
        y_intra = lax.dot_general(u_d, pt, (((1,), (0,)), ((), ())),
                                  preferred_element_type=_F32, precision=_HI)
        contrib = lax.dot_general(u_d, revp, (((1,), (0,)), ((), ())),
                                  preferred_element_type=_F32, precision=_HI)
        s = contrib
        w = rT
        for k in (1, 2, 4, 8):
            sh = jnp.concatenate([s[BC_ - k:, :], s[:BC_ - k, :]], axis=0)
            s = s + w * jnp.where(gi >= k, sh, 0.0)
            w = w * w
        hprev = jnp.concatenate([s[BC_ - 1:, :], s[:BC_ - 1, :]], axis=0)
        hprev = jnp.where(gi >= 1, hprev, 0.0)
        y_outer = lax.dot_general(hprev, wop, (((1,), (1,)), ((), ())),
                                  preferred_element_type=_F32, precision=_HI)
        o_ref[i] = y_intra + y_outer + dd_ref[i:i + 1, :] * u_d


def _outproj_kernel(y_ref, w_ref, b_ref, o_ref):
    y = y_ref[...]
    w = w_ref[...]
    z = lax.dot_general(y, w, (((0,), (1,)), ((), ())),
                        preferred_element_type=_F32)
    o_ref[...] = (z + b_ref[...])[None]


def kernel(x, W_in, b_in, A_log, B, C, D, dt, W_out, b_out):
    b_in2 = b_in.reshape(D_, 1)
    dt2 = dt.reshape(D_, 1)
    dd2 = D.reshape(D_, 1)
    b_out2 = b_out.reshape(1, D_)

    u = pl.pallas_call(
        _inproj_kernel,
        grid=(D_ // DT1, B_, C_),
        in_specs=[
            pl.BlockSpec((1, T_, D_), lambda di, b, c: (b, c, 0)),
            pl.BlockSpec((DT1, D_), lambda di, b, c: (di, 0)),
            pl.BlockSpec((DT1, 1), lambda di, b, c: (di, 0)),
        ],
        out_specs=pl.BlockSpec((DT1, T_), lambda di, b, c: (di, b * C_ + c)),
        out_shape=jax.ShapeDtypeStruct((D_, B_ * L_), _F32),
        compiler_params=pltpu.CompilerParams(
            dimension_semantics=("parallel", "arbitrary", "arbitrary")),
        name="s4_inproj",
    )(x, W_in, b_in2)

    u3 = u.reshape(D_, BC_, T_)
    yconv = pl.pallas_call(
        _conv_kernel,
        grid=(D_ // DTC,),
        in_specs=[
            pl.BlockSpec((DTC, BC_, T_), lambda dj: (dj, 0, 0)),
            pl.BlockSpec((DTC, N_), lambda dj: (dj, 0)),
            pl.BlockSpec((DTC, N_), lambda dj: (dj, 0)),
            pl.BlockSpec((DTC, N_), lambda dj: (dj, 0)),
            pl.BlockSpec((DTC, 1), lambda dj: (dj, 0)),
            pl.BlockSpec((DTC, 1), lambda dj: (dj, 0)),
        ],
        out_specs=pl.BlockSpec((DTC, BC_, T_), lambda dj: (dj, 0, 0)),
        out_shape=jax.ShapeDtypeStruct((D_, BC_, T_), _F32),
        compiler_params=pltpu.CompilerParams(
            dimension_semantics=("parallel",)),
        name="s4_ssm_conv",
    )(u3, A_log, B, C, dt2, dd2)

    y2 = yconv.reshape(D_, B_ * L_)
    out = pl.pallas_call(
        _outproj_kernel,
        grid=(D_ // DTO, B_, C_),
        in_specs=[
            pl.BlockSpec((D_, T_), lambda do, b, c: (0, b * C_ + c)),
            pl.BlockSpec((DTO, D_), lambda do, b, c: (do, 0)),
            pl.BlockSpec((1, DTO), lambda do, b, c: (0, do)),
        ],
        out_specs=pl.BlockSpec((1, T_, DTO), lambda do, b, c: (b, c, do)),
        out_shape=jax.ShapeDtypeStruct((B_, L_, D_), _F32),
        compiler_params=pltpu.CompilerParams(
            dimension_semantics=("parallel", "arbitrary", "arbitrary")),
        name="s4_outproj",
    )(y2, W_out, b_out2)
    return out


# speedup vs baseline: 5.5641x; 1.7184x over previous
import jax
import jax.numpy as jnp
from jax import lax
from jax.experimental import pallas as pl
from jax.experimental.pallas import tpu as pltpu

B_ = 4
L_ = 2048
D_ = 512
N_ = 64
T_ = 128
C_ = L_ // T_
BC_ = B_ * C_
DT1 = 256
DTC = 8
DTO = 256

_F32 = jnp.float32


def _inproj_kernel(x_ref, w_ref, b_ref, o_ref):
    xt = x_ref[0]
    w = w_ref[...]
    acc = lax.dot_general(w, xt, (((1,), (1,)), ((), ())),
                          preferred_element_type=_F32)
    o_ref[...] = (acc + b_ref[...])[:, None, None, :]


def _conv_kernel(u_ref, a_ref, bb_ref, cc_ref, dt_ref, dd_ref, o_ref):
    tau = lax.broadcasted_iota(jnp.int32, (T_, N_), 0).astype(_F32)
    row_i = lax.broadcasted_iota(jnp.int32, (T_, T_), 0)
    col_i = lax.broadcasted_iota(jnp.int32, (T_, T_), 1)
    causal = col_i >= row_i
    gi = lax.broadcasted_iota(jnp.int32, (BC_, N_), 0) % C_

    for i in range(DTC):
        a_row = a_ref[i:i + 1, :]
        dt_d = dt_ref[i:i + 1, :]
        logr = -jnp.exp(a_row) * dt_d
        coef = bb_ref[i:i + 1, :] * cc_ref[i:i + 1, :] * dt_d
        pows = jnp.exp(tau * logr)
        r1 = jnp.exp(logr)
        rT = jnp.exp(float(T_) * logr)
        rTm1 = rT * (1.0 / r1)
        rneg = 1.0 / pows
        revp = rTm1 * rneg
        cp = coef * pows
        wop = cp * r1
        u_d = u_ref[i].reshape(BC_, T_)

        pt = lax.dot_general(rneg, cp, (((1,), (1,)), ((), ())),
                             preferred_element_type=_F32)
        pt = jnp.where(causal, pt, 0.0)
        y_intra = lax.dot_general(u_d, pt, (((1,), (0,)), ((), ())),
                                  preferred_element_type=_F32)
        contrib = lax.dot_general(u_d, revp, (((1,), (0,)), ((), ())),
                                  preferred_element_type=_F32)
        s = contrib
        w = rT
        for k in (1, 2, 4, 8):
            sh = jnp.concatenate([s[BC_ - k:, :], s[:BC_ - k, :]], axis=0)
            s = s + w * jnp.where(gi >= k, sh, 0.0)
            w = w * w
        hprev = jnp.concatenate([s[BC_ - 1:, :], s[:BC_ - 1, :]], axis=0)
        hprev = jnp.where(gi >= 1, hprev, 0.0)
        y_outer = lax.dot_general(hprev, wop, (((1,), (1,)), ((), ())),
                                  preferred_element_type=_F32)
        y = y_intra + y_outer + dd_ref[i:i + 1, :] * u_d
        o_ref[i] = y[:, None, :]


def _outproj_kernel(y_ref, w_ref, b_ref, o_ref):
    y = y_ref[...].reshape(D_, T_)
    w = w_ref[...]
    z = lax.dot_general(y, w, (((0,), (1,)), ((), ())),
                        preferred_element_type=_F32)
    o_ref[...] = (z + b_ref[...])[None]


def kernel(x, W_in, b_in, A_log, B, C, D, dt, W_out, b_out):
    b_in2 = b_in.reshape(D_, 1)
    dt2 = dt.reshape(D_, 1)
    dd2 = D.reshape(D_, 1)
    b_out2 = b_out.reshape(1, D_)

    u = pl.pallas_call(
        _inproj_kernel,
        grid=(D_ // DT1, B_, C_),
        in_specs=[
            pl.BlockSpec((1, T_, D_), lambda di, b, c: (b, c, 0)),
            pl.BlockSpec((DT1, D_), lambda di, b, c: (di, 0)),
            pl.BlockSpec((DT1, 1), lambda di, b, c: (di, 0)),
        ],
        out_specs=pl.BlockSpec((DT1, 1, 1, T_),
                               lambda di, b, c: (di, b * C_ + c, 0, 0)),
        out_shape=jax.ShapeDtypeStruct((D_, BC_, 1, T_), _F32),
        compiler_params=pltpu.CompilerParams(
            dimension_semantics=("parallel", "arbitrary", "arbitrary")),
        name="s4_inproj",
    )(x, W_in, b_in2)

    yconv = pl.pallas_call(
        _conv_kernel,
        grid=(D_ // DTC,),
        in_specs=[
            pl.BlockSpec((DTC, BC_, 1, T_), lambda dj: (dj, 0, 0, 0)),
            pl.BlockSpec((DTC, N_), lambda dj: (dj, 0)),
            pl.BlockSpec((DTC, N_), lambda dj: (dj, 0)),
            pl.BlockSpec((DTC, N_), lambda dj: (dj, 0)),
            pl.BlockSpec((DTC, 1), lambda dj: (dj, 0)),
            pl.BlockSpec((DTC, 1), lambda dj: (dj, 0)),
        ],
        out_specs=pl.BlockSpec((DTC, BC_, 1, T_), lambda dj: (dj, 0, 0, 0)),
        out_shape=jax.ShapeDtypeStruct((D_, BC_, 1, T_), _F32),
        compiler_params=pltpu.CompilerParams(
            dimension_semantics=("parallel",)),
        name="s4_ssm_conv",
    )(u, A_log, B, C, dt2, dd2)

    out = pl.pallas_call(
        _outproj_kernel,
        grid=(D_ // DTO, B_, C_),
        in_specs=[
            pl.BlockSpec((D_, 1, 1, T_), lambda do, b, c: (0, b * C_ + c, 0, 0)),
            pl.BlockSpec((DTO, D_), lambda do, b, c: (do, 0)),
            pl.BlockSpec((1, DTO), lambda do, b, c: (0, do)),
        ],
        out_specs=pl.BlockSpec((1, T_, DTO), lambda do, b, c: (b, c, do)),
        out_shape=jax.ShapeDtypeStruct((B_, L_, D_), _F32),
        compiler_params=pltpu.CompilerParams(
            dimension_semantics=("parallel", "arbitrary", "arbitrary")),
        name="s4_outproj",
    )(yconv, W_out, b_out2)
    return out


# speedup vs baseline: 9.5179x; 1.7106x over previous
import jax
import jax.numpy as jnp
from jax import lax
from jax.experimental import pallas as pl
from jax.experimental.pallas import tpu as pltpu

B_ = 4
L_ = 2048
D_ = 512
N_ = 64
T_ = 128
C_ = L_ // T_
BC_ = B_ * C_
DT1 = 256
DTC = 8
DTO = 256

_F32 = jnp.float32


def _inproj_kernel(x_ref, w_ref, b_ref, o_ref):
    xt = x_ref[0]
    w = w_ref[...]
    acc = lax.dot_general(w, xt, (((1,), (1,)), ((), ())),
                          preferred_element_type=_F32)
    o_ref[...] = acc + b_ref[...]


def _conv_kernel(u_ref, a_ref, bb_ref, cc_ref, dt_ref, dd_ref, o_ref):
    tau = lax.broadcasted_iota(jnp.int32, (T_, N_), 0).astype(_F32)
    row_i = lax.broadcasted_iota(jnp.int32, (T_, T_), 0)
    col_i = lax.broadcasted_iota(jnp.int32, (T_, T_), 1)
    causal = col_i >= row_i
    gi = lax.broadcasted_iota(jnp.int32, (BC_, N_), 0) % C_

    for i in range(DTC):
        a_row = a_ref[i:i + 1, :]
        dt_d = dt_ref[i:i + 1, :]
        logr = -jnp.exp(a_row) * dt_d
        coef = bb_ref[i:i + 1, :] * cc_ref[i:i + 1, :] * dt_d
        pows = jnp.exp(tau * logr)
        r1 = jnp.exp(logr)
        rT = jnp.exp(float(T_) * logr)
        rTm1 = rT * (1.0 / r1)
        rneg = 1.0 / pows
        revp = rTm1 * rneg
        cp = coef * pows
        wop = cp * r1
        u_d = u_ref[i]

        pt = lax.dot_general(rneg, cp, (((1,), (1,)), ((), ())),
                             preferred_element_type=_F32)
        pt = jnp.where(causal, pt, 0.0)
        y_intra = lax.dot_general(u_d, pt, (((1,), (0,)), ((), ())),
                                  preferred_element_type=_F32)
        contrib = lax.dot_general(u_d, revp, (((1,), (0,)), ((), ())),
                                  preferred_element_type=_F32)
        s = contrib
        w = rT
        for k in (1, 2, 4, 8):
            sh = jnp.concatenate([s[BC_ - k:, :], s[:BC_ - k, :]], axis=0)
            s = s + w * jnp.where(gi >= k, sh, 0.0)
            w = w * w
        hprev = jnp.concatenate([s[BC_ - 1:, :], s[:BC_ - 1, :]], axis=0)
        hprev = jnp.where(gi >= 1, hprev, 0.0)
        y_outer = lax.dot_general(hprev, wop, (((1,), (1,)), ((), ())),
                                  preferred_element_type=_F32)
        o_ref[i] = y_intra + y_outer + dd_ref[i:i + 1, :] * u_d


def _outproj_kernel(y_ref, w_ref, b_ref, o_ref):
    y = y_ref[...]
    w = w_ref[...]
    z = lax.dot_general(y, w, (((0,), (1,)), ((), ())),
                        preferred_element_type=_F32)
    o_ref[...] = (z + b_ref[...])[None]


def kernel(x, W_in, b_in, A_log, B, C, D, dt, W_out, b_out):
    b_in2 = b_in.reshape(D_, 1)
    dt2 = dt.reshape(D_, 1)
    dd2 = D.reshape(D_, 1)
    b_out2 = b_out.reshape(1, D_)

    u = pl.pallas_call(
        _inproj_kernel,
        grid=(D_ // DT1, B_),
        in_specs=[
            pl.BlockSpec((1, L_, D_), lambda di, b: (b, 0, 0)),
            pl.BlockSpec((DT1, D_), lambda di, b: (di, 0)),
            pl.BlockSpec((DT1, 1), lambda di, b: (di, 0)),
        ],
        out_specs=pl.BlockSpec((DT1, L_), lambda di, b: (di, b)),
        out_shape=jax.ShapeDtypeStruct((D_, B_ * L_), _F32),
        compiler_params=pltpu.CompilerParams(
            dimension_semantics=("parallel", "arbitrary")),
        name="s4_inproj",
    )(x, W_in, b_in2)

    u3 = u.reshape(D_, BC_, T_)
    yconv = pl.pallas_call(
        _conv_kernel,
        grid=(D_ // DTC,),
        in_specs=[
            pl.BlockSpec((DTC, BC_, T_), lambda dj: (dj, 0, 0)),
            pl.BlockSpec((DTC, N_), lambda dj: (dj, 0)),
            pl.BlockSpec((DTC, N_), lambda dj: (dj, 0)),
            pl.BlockSpec((DTC, N_), lambda dj: (dj, 0)),
            pl.BlockSpec((DTC, 1), lambda dj: (dj, 0)),
            pl.BlockSpec((DTC, 1), lambda dj: (dj, 0)),
        ],
        out_specs=pl.BlockSpec((DTC, BC_, T_), lambda dj: (dj, 0, 0)),
        out_shape=jax.ShapeDtypeStruct((D_, BC_, T_), _F32),
        compiler_params=pltpu.CompilerParams(
            dimension_semantics=("parallel",)),
        name="s4_ssm_conv",
    )(u3, A_log, B, C, dt2, dd2)

    y2 = yconv.reshape(D_, B_ * L_)
    out = pl.pallas_call(
        _outproj_kernel,
        grid=(D_ // DTO, B_),
        in_specs=[
            pl.BlockSpec((D_, L_), lambda do, b: (0, b)),
            pl.BlockSpec((DTO, D_), lambda do, b: (do, 0)),
            pl.BlockSpec((1, DTO), lambda do, b: (0, do)),
        ],
        out_specs=pl.BlockSpec((1, L_, DTO), lambda do, b: (b, 0, do)),
        out_shape=jax.ShapeDtypeStruct((B_, L_, D_), _F32),
        compiler_params=pltpu.CompilerParams(
            dimension_semantics=("parallel", "arbitrary")),
        name="s4_outproj",
    )(y2, W_out, b_out2)
    return out


# speedup vs baseline: 11.2860x; 1.1858x over previous
import jax
import jax.numpy as jnp
from jax import lax
from jax.experimental import pallas as pl
from jax.experimental.pallas import tpu as pltpu

B_ = 4
L_ = 2048
D_ = 512
N_ = 64
T_ = 128
C_ = L_ // T_
BC_ = B_ * C_
DT1 = 256
DTC = 32
DTO = 256

_F32 = jnp.float32


def _inproj_kernel(x_ref, w_ref, b_ref, o_ref):
    xt = x_ref[0]
    w = w_ref[...]
    acc = lax.dot_general(w, xt, (((1,), (1,)), ((), ())),
                          preferred_element_type=_F32)
    o_ref[...] = acc + b_ref[...]


def _conv_kernel(u_ref, a_ref, bb_ref, cc_ref, dt_ref, dd_ref, o_ref):
    tau = lax.broadcasted_iota(jnp.int32, (T_, N_), 0).astype(_F32)
    row_i = lax.broadcasted_iota(jnp.int32, (T_, T_), 0)
    col_i = lax.broadcasted_iota(jnp.int32, (T_, T_), 1)
    causal = col_i >= row_i
    gi = lax.broadcasted_iota(jnp.int32, (BC_, N_), 0) % C_

    for i in range(DTC):
        a_row = a_ref[i:i + 1, :]
        dt_d = dt_ref[i:i + 1, :]
        logr = -jnp.exp(a_row) * dt_d
        coef = bb_ref[i:i + 1, :] * cc_ref[i:i + 1, :] * dt_d
        pows = jnp.exp(tau * logr)
        r1 = jnp.exp(logr)
        rT = jnp.exp(float(T_) * logr)
        rTm1 = rT * (1.0 / r1)
        rneg = 1.0 / pows
        revp = rTm1 * rneg
        cp = coef * pows
        wop = cp * r1
        u_d = u_ref[i]

        pt = lax.dot_general(rneg, cp, (((1,), (1,)), ((), ())),
                             preferred_element_type=_F32)
        pt = jnp.where(causal, pt, 0.0)
        y_intra = lax.dot_general(u_d, pt, (((1,), (0,)), ((), ())),
                                  preferred_element_type=_F32)
        contrib = lax.dot_general(u_d, revp, (((1,), (0,)), ((), ())),
                                  preferred_element_type=_F32)
        s = contrib
        w = rT
        for k in (1, 2, 4, 8):
            sh = jnp.concatenate([s[BC_ - k:, :], s[:BC_ - k, :]], axis=0)
            s = s + w * jnp.where(gi >= k, sh, 0.0)
            w = w * w
        hprev = jnp.concatenate([s[BC_ - 1:, :], s[:BC_ - 1, :]], axis=0)
        hprev = jnp.where(gi >= 1, hprev, 0.0)
        y_outer = lax.dot_general(hprev, wop, (((1,), (1,)), ((), ())),
                                  preferred_element_type=_F32)
        o_ref[i] = y_intra + y_outer + dd_ref[i:i + 1, :] * u_d


def _outproj_kernel(y_ref, w_ref, b_ref, o_ref):
    y = y_ref[...]
    w = w_ref[...]
    z = lax.dot_general(y, w, (((0,), (1,)), ((), ())),
                        preferred_element_type=_F32)
    o_ref[...] = (z + b_ref[...])[None]


def kernel(x, W_in, b_in, A_log, B, C, D, dt, W_out, b_out):
    b_in2 = b_in.reshape(D_, 1)
    dt2 = dt.reshape(D_, 1)
    dd2 = D.reshape(D_, 1)
    b_out2 = b_out.reshape(1, D_)

    u = pl.pallas_call(
        _inproj_kernel,
        grid=(D_ // DT1, B_),
        in_specs=[
            pl.BlockSpec((1, L_, D_), lambda di, b: (b, 0, 0)),
            pl.BlockSpec((DT1, D_), lambda di, b: (di, 0)),
            pl.BlockSpec((DT1, 1), lambda di, b: (di, 0)),
        ],
        out_specs=pl.BlockSpec((DT1, L_), lambda di, b: (di, b)),
        out_shape=jax.ShapeDtypeStruct((D_, B_ * L_), _F32),
        compiler_params=pltpu.CompilerParams(
            dimension_semantics=("parallel", "arbitrary")),
        name="s4_inproj",
    )(x, W_in, b_in2)

    u3 = u.reshape(D_, BC_, T_)
    yconv = pl.pallas_call(
        _conv_kernel,
        grid=(D_ // DTC,),
        in_specs=[
            pl.BlockSpec((DTC, BC_, T_), lambda dj: (dj, 0, 0)),
            pl.BlockSpec((DTC, N_), lambda dj: (dj, 0)),
            pl.BlockSpec((DTC, N_), lambda dj: (dj, 0)),
            pl.BlockSpec((DTC, N_), lambda dj: (dj, 0)),
            pl.BlockSpec((DTC, 1), lambda dj: (dj, 0)),
            pl.BlockSpec((DTC, 1), lambda dj: (dj, 0)),
        ],
        out_specs=pl.BlockSpec((DTC, BC_, T_), lambda dj: (dj, 0, 0)),
        out_shape=jax.ShapeDtypeStruct((D_, BC_, T_), _F32),
        compiler_params=pltpu.CompilerParams(
            dimension_semantics=("parallel",)),
        name="s4_ssm_conv",
    )(u3, A_log, B, C, dt2, dd2)

    y2 = yconv.reshape(D_, B_ * L_)
    out = pl.pallas_call(
        _outproj_kernel,
        grid=(D_ // DTO, B_),
        in_specs=[
            pl.BlockSpec((D_, L_), lambda do, b: (0, b)),
            pl.BlockSpec((DTO, D_), lambda do, b: (do, 0)),
            pl.BlockSpec((1, DTO), lambda do, b: (0, do)),
        ],
        out_specs=pl.BlockSpec((1, L_, DTO), lambda do, b: (b, 0, do)),
        out_shape=jax.ShapeDtypeStruct((B_, L_, D_), _F32),
        compiler_params=pltpu.CompilerParams(
            dimension_semantics=("parallel", "arbitrary")),
        name="s4_outproj",
    )(y2, W_out, b_out2)
    return out


# speedup vs baseline: 11.2996x; 1.0012x over previous
import jax
import jax.numpy as jnp
from jax import lax
from jax.experimental import pallas as pl
from jax.experimental.pallas import tpu as pltpu

B_ = 4
L_ = 2048
D_ = 512
N_ = 64
T_ = 128
C_ = L_ // T_
BC_ = B_ * C_
DT1 = 256
DTC = 32
DTO = 256

_F32 = jnp.float32


def _inproj_kernel(x_ref, w_ref, b_ref, o_ref):
    xt = x_ref[0]
    w = w_ref[...]
    acc = lax.dot_general(w, xt, (((1,), (1,)), ((), ())),
                          preferred_element_type=_F32)
    o_ref[...] = acc + b_ref[...]


def _conv_kernel(u_ref, a_ref, bb_ref, cc_ref, dt_ref, dd_ref, o_ref):
    tau = lax.broadcasted_iota(jnp.int32, (T_, N_), 0).astype(_F32)
    row_i = lax.broadcasted_iota(jnp.int32, (T_, T_), 0)
    col_i = lax.broadcasted_iota(jnp.int32, (T_, T_), 1)
    causal = col_i >= row_i
    gi = lax.broadcasted_iota(jnp.int32, (BC_, N_), 0) % C_

    logr_all = -jnp.exp(a_ref[...]) * dt_ref[...]
    coef_all = bb_ref[...] * cc_ref[...] * dt_ref[...]
    r1_all = jnp.exp(logr_all)
    rT_all = jnp.exp(float(T_) * logr_all)
    rTm1_all = rT_all * (1.0 / r1_all)

    for i in range(DTC):
        logr = logr_all[i:i + 1, :]
        coef = coef_all[i:i + 1, :]
        r1 = r1_all[i:i + 1, :]
        rT = rT_all[i:i + 1, :]
        pows = jnp.exp(tau * logr)
        rneg = 1.0 / pows
        revp = rTm1_all[i:i + 1, :] * rneg
        cp = coef * pows
        wop = cp * r1
        u_d = u_ref[i]

        pt = lax.dot_general(rneg, cp, (((1,), (1,)), ((), ())),
                             preferred_element_type=_F32)
        pt = jnp.where(causal, pt, 0.0)
        y_intra = lax.dot_general(u_d, pt, (((1,), (0,)), ((), ())),
                                  preferred_element_type=_F32)
        contrib = lax.dot_general(u_d, revp, (((1,), (0,)), ((), ())),
                                  preferred_element_type=_F32)
        s = contrib
        w = rT
        for k in (1, 2, 4, 8):
            sh = jnp.concatenate([s[BC_ - k:, :], s[:BC_ - k, :]], axis=0)
            s = s + w * jnp.where(gi >= k, sh, 0.0)
            w = w * w
        hprev = jnp.concatenate([s[BC_ - 1:, :], s[:BC_ - 1, :]], axis=0)
        hprev = jnp.where(gi >= 1, hprev, 0.0)
        y_outer = lax.dot_general(hprev, wop, (((1,), (1,)), ((), ())),
                                  preferred_element_type=_F32)
        o_ref[i] = y_intra + y_outer + dd_ref[i:i + 1, :] * u_d


def _outproj_kernel(y_ref, w_ref, b_ref, o_ref):
    y = y_ref[...]
    w = w_ref[...]
    z = lax.dot_general(y, w, (((0,), (1,)), ((), ())),
                        preferred_element_type=_F32)
    o_ref[...] = (z + b_ref[...])[None]


def kernel(x, W_in, b_in, A_log, B, C, D, dt, W_out, b_out):
    b_in2 = b_in.reshape(D_, 1)
    dt2 = dt.reshape(D_, 1)
    dd2 = D.reshape(D_, 1)
    b_out2 = b_out.reshape(1, D_)

    u = pl.pallas_call(
        _inproj_kernel,
        grid=(D_ // DT1, B_),
        in_specs=[
            pl.BlockSpec((1, L_, D_), lambda di, b: (b, 0, 0)),
            pl.BlockSpec((DT1, D_), lambda di, b: (di, 0)),
            pl.BlockSpec((DT1, 1), lambda di, b: (di, 0)),
        ],
        out_specs=pl.BlockSpec((DT1, L_), lambda di, b: (di, b)),
        out_shape=jax.ShapeDtypeStruct((D_, B_ * L_), _F32),
        compiler_params=pltpu.CompilerParams(
            dimension_semantics=("parallel", "arbitrary")),
        name="s4_inproj",
    )(x, W_in, b_in2)

    u3 = u.reshape(D_, BC_, T_)
    yconv = pl.pallas_call(
        _conv_kernel,
        grid=(D_ // DTC,),
        in_specs=[
            pl.BlockSpec((DTC, BC_, T_), lambda dj: (dj, 0, 0)),
            pl.BlockSpec((DTC, N_), lambda dj: (dj, 0)),
            pl.BlockSpec((DTC, N_), lambda dj: (dj, 0)),
            pl.BlockSpec((DTC, N_), lambda dj: (dj, 0)),
            pl.BlockSpec((DTC, 1), lambda dj: (dj, 0)),
            pl.BlockSpec((DTC, 1), lambda dj: (dj, 0)),
        ],
        out_specs=pl.BlockSpec((DTC, BC_, T_), lambda dj: (dj, 0, 0)),
        out_shape=jax.ShapeDtypeStruct((D_, BC_, T_), _F32),
        compiler_params=pltpu.CompilerParams(
            dimension_semantics=("parallel",)),
        name="s4_ssm_conv",
    )(u3, A_log, B, C, dt2, dd2)

    y2 = yconv.reshape(D_, B_ * L_)
    out = pl.pallas_call(
        _outproj_kernel,
        grid=(D_ // DTO, B_),
        in_specs=[
            pl.BlockSpec((D_, L_), lambda do, b: (0, b)),
            pl.BlockSpec((DTO, D_), lambda do, b: (do, 0)),
            pl.BlockSpec((1, DTO), lambda do, b: (0, do)),
        ],
        out_specs=pl.BlockSpec((1, L_, DTO), lambda do, b: (b, 0, do)),
        out_shape=jax.ShapeDtypeStruct((B_, L_, D_), _F32),
        compiler_params=pltpu.CompilerParams(
            dimension_semantics=("parallel", "arbitrary")),
        name="s4_outproj",
    )(y2, W_out, b_out2)
    return out


# speedup vs baseline: 12.6361x; 1.1183x over previous
import jax
import jax.numpy as jnp
from jax import lax
from jax.experimental import pallas as pl
from jax.experimental.pallas import tpu as pltpu

B_ = 4
L_ = 2048
D_ = 512
N_ = 64
T_ = 128
C_ = L_ // T_
BC_ = B_ * C_
DT1 = 256
DTC = 32
DTO = 256

_F32 = jnp.float32


def _inproj_kernel(x_ref, w_ref, b_ref, o_ref):
    xt = x_ref[0]
    w = w_ref[...]
    acc = lax.dot_general(w, xt, (((1,), (1,)), ((), ())),
                          preferred_element_type=_F32) + b_ref[...]
    for c in range(C_):
        o_ref[:, c, :] = acc[:, c * T_:(c + 1) * T_]


def _conv_kernel(u_ref, a_ref, bb_ref, cc_ref, dt_ref, dd_ref, o_ref):
    tau = lax.broadcasted_iota(jnp.int32, (T_, N_), 0).astype(_F32)
    row_i = lax.broadcasted_iota(jnp.int32, (T_, T_), 0)
    col_i = lax.broadcasted_iota(jnp.int32, (T_, T_), 1)
    causal = col_i >= row_i
    gi = lax.broadcasted_iota(jnp.int32, (BC_, N_), 0) % C_

    logr_all = -jnp.exp(a_ref[...]) * dt_ref[...]
    coef_all = bb_ref[...] * cc_ref[...] * dt_ref[...]
    r1_all = jnp.exp(logr_all)
    rT_all = jnp.exp(float(T_) * logr_all)
    rTm1_all = rT_all * (1.0 / r1_all)

    for i in range(DTC):
        logr = logr_all[i:i + 1, :]
        coef = coef_all[i:i + 1, :]
        r1 = r1_all[i:i + 1, :]
        rT = rT_all[i:i + 1, :]
        pows = jnp.exp(tau * logr)
        rneg = 1.0 / pows
        revp = rTm1_all[i:i + 1, :] * rneg
        cp = coef * pows
        wop = cp * r1
        u_d = u_ref[i]

        pt = lax.dot_general(rneg, cp, (((1,), (1,)), ((), ())),
                             preferred_element_type=_F32)
        pt = jnp.where(causal, pt, 0.0)
        y_intra = lax.dot_general(u_d, pt, (((1,), (0,)), ((), ())),
                                  preferred_element_type=_F32)
        contrib = lax.dot_general(u_d, revp, (((1,), (0,)), ((), ())),
                                  preferred_element_type=_F32)
        s = contrib
        w = rT
        for k in (1, 2, 4, 8):
            sh = jnp.concatenate([s[BC_ - k:, :], s[:BC_ - k, :]], axis=0)
            s = s + w * jnp.where(gi >= k, sh, 0.0)
            w = w * w
        hprev = jnp.concatenate([s[BC_ - 1:, :], s[:BC_ - 1, :]], axis=0)
        hprev = jnp.where(gi >= 1, hprev, 0.0)
        y_outer = lax.dot_general(hprev, wop, (((1,), (1,)), ((), ())),
                                  preferred_element_type=_F32)
        y = y_intra + y_outer + dd_ref[i:i + 1, :] * u_d
        o_ref[i] = y.astype(jnp.bfloat16)


def _outproj_kernel(y_ref, w_ref, b_ref, o_ref):
    y = y_ref[...]
    w = w_ref[...]
    z = lax.dot_general(y, w, (((0,), (1,)), ((), ())),
                        preferred_element_type=_F32)
    o_ref[...] = (z + b_ref[...])[None]


def kernel(x, W_in, b_in, A_log, B, C, D, dt, W_out, b_out):
    b_in2 = b_in.reshape(D_, 1)
    dt2 = dt.reshape(D_, 1)
    dd2 = D.reshape(D_, 1)
    b_out2 = b_out.reshape(1, D_)

    u = pl.pallas_call(
        _inproj_kernel,
        grid=(D_ // DT1, B_),
        in_specs=[
            pl.BlockSpec((1, L_, D_), lambda di, b: (b, 0, 0)),
            pl.BlockSpec((DT1, D_), lambda di, b: (di, 0)),
            pl.BlockSpec((DT1, 1), lambda di, b: (di, 0)),
        ],
        out_specs=pl.BlockSpec((DT1, C_, T_), lambda di, b: (di, b, 0)),
        out_shape=jax.ShapeDtypeStruct((D_, BC_, T_), _F32),
        compiler_params=pltpu.CompilerParams(
            dimension_semantics=("parallel", "arbitrary")),
        name="s4_inproj",
    )(x, W_in, b_in2)

    u3 = u
    yconv = pl.pallas_call(
        _conv_kernel,
        grid=(D_ // DTC,),
        in_specs=[
            pl.BlockSpec((DTC, BC_, T_), lambda dj: (dj, 0, 0)),
            pl.BlockSpec((DTC, N_), lambda dj: (dj, 0)),
            pl.BlockSpec((DTC, N_), lambda dj: (dj, 0)),
            pl.BlockSpec((DTC, N_), lambda dj: (dj, 0)),
            pl.BlockSpec((DTC, 1), lambda dj: (dj, 0)),
            pl.BlockSpec((DTC, 1), lambda dj: (dj, 0)),
        ],
        out_specs=pl.BlockSpec((DTC, BC_, T_), lambda dj: (dj, 0, 0)),
        out_shape=jax.ShapeDtypeStruct((D_, BC_, T_), jnp.bfloat16),
        compiler_params=pltpu.CompilerParams(
            dimension_semantics=("parallel",)),
        name="s4_ssm_conv",
    )(u3, A_log, B, C, dt2, dd2)

    y2 = yconv.reshape(D_, B_ * L_)
    out = pl.pallas_call(
        _outproj_kernel,
        grid=(D_ // DTO, B_),
        in_specs=[
            pl.BlockSpec((D_, L_), lambda do, b: (0, b)),
            pl.BlockSpec((DTO, D_), lambda do, b: (do, 0)),
            pl.BlockSpec((1, DTO), lambda do, b: (0, do)),
        ],
        out_specs=pl.BlockSpec((1, L_, DTO), lambda do, b: (b, 0, do)),
        out_shape=jax.ShapeDtypeStruct((B_, L_, D_), _F32),
        compiler_params=pltpu.CompilerParams(
            dimension_semantics=("parallel", "arbitrary")),
        name="s4_outproj",
    )(y2, W_out, b_out2)
    return out


# speedup vs baseline: 12.9583x; 1.0255x over previous
import jax
import jax.numpy as jnp
from jax import lax
from jax.experimental import pallas as pl
from jax.experimental.pallas import tpu as pltpu

B_ = 4
L_ = 2048
D_ = 512
N_ = 64
T_ = 128
C_ = L_ // T_
BC_ = B_ * C_
DT1 = 256
DTC = 64
DTO = 256

_F32 = jnp.float32


def _inproj_kernel(x_ref, w_ref, b_ref, o_ref):
    xt = x_ref[0]
    w = w_ref[...]
    acc = lax.dot_general(w, xt, (((1,), (1,)), ((), ())),
                          preferred_element_type=_F32) + b_ref[...]
    for c in range(C_):
        o_ref[:, c, :] = acc[:, c * T_:(c + 1) * T_]


def _conv_kernel(u_ref, a_ref, bb_ref, cc_ref, dt_ref, dd_ref, o_ref):
    tau = lax.broadcasted_iota(jnp.int32, (T_, N_), 0).astype(_F32)
    row_i = lax.broadcasted_iota(jnp.int32, (T_, T_), 0)
    col_i = lax.broadcasted_iota(jnp.int32, (T_, T_), 1)
    causal = col_i >= row_i
    gi = lax.broadcasted_iota(jnp.int32, (BC_, N_), 0) % C_

    logr_all = -jnp.exp(a_ref[...]) * dt_ref[...]
    coef_all = bb_ref[...] * cc_ref[...] * dt_ref[...]
    r1_all = jnp.exp(logr_all)
    rT_all = jnp.exp(float(T_) * logr_all)
    rTm1_all = rT_all * (1.0 / r1_all)

    for i in range(DTC):
        logr = logr_all[i:i + 1, :]
        coef = coef_all[i:i + 1, :]
        r1 = r1_all[i:i + 1, :]
        rT = rT_all[i:i + 1, :]
        pows = jnp.exp(tau * logr)
        rneg = 1.0 / pows
        revp = rTm1_all[i:i + 1, :] * rneg
        cp = coef * pows
        wop = cp * r1
        u_d = u_ref[i]

        pt = lax.dot_general(rneg, cp, (((1,), (1,)), ((), ())),
                             preferred_element_type=_F32)
        pt = jnp.where(causal, pt, 0.0)
        y_intra = lax.dot_general(u_d, pt, (((1,), (0,)), ((), ())),
                                  preferred_element_type=_F32)
        contrib = lax.dot_general(u_d, revp, (((1,), (0,)), ((), ())),
                                  preferred_element_type=_F32)
        s = contrib
        w = rT
        for k in (1, 2, 4, 8):
            sh = jnp.concatenate([s[BC_ - k:, :], s[:BC_ - k, :]], axis=0)
            s = s + w * jnp.where(gi >= k, sh, 0.0)
            w = w * w
        hprev = jnp.concatenate([s[BC_ - 1:, :], s[:BC_ - 1, :]], axis=0)
        hprev = jnp.where(gi >= 1, hprev, 0.0)
        y_outer = lax.dot_general(hprev, wop, (((1,), (1,)), ((), ())),
                                  preferred_element_type=_F32)
        y = y_intra + y_outer + dd_ref[i:i + 1, :] * u_d
        o_ref[i] = y.astype(jnp.bfloat16)


def _outproj_kernel(y_ref, w_ref, b_ref, o_ref):
    y = y_ref[...]
    w = w_ref[...]
    z = lax.dot_general(y, w, (((0,), (1,)), ((), ())),
                        preferred_element_type=_F32)
    o_ref[...] = (z + b_ref[...])[None]


def kernel(x, W_in, b_in, A_log, B, C, D, dt, W_out, b_out):
    b_in2 = b_in.reshape(D_, 1)
    dt2 = dt.reshape(D_, 1)
    dd2 = D.reshape(D_, 1)
    b_out2 = b_out.reshape(1, D_)

    u = pl.pallas_call(
        _inproj_kernel,
        grid=(D_ // DT1, B_),
        in_specs=[
            pl.BlockSpec((1, L_, D_), lambda di, b: (b, 0, 0)),
            pl.BlockSpec((DT1, D_), lambda di, b: (di, 0)),
            pl.BlockSpec((DT1, 1), lambda di, b: (di, 0)),
        ],
        out_specs=pl.BlockSpec((DT1, C_, T_), lambda di, b: (di, b, 0)),
        out_shape=jax.ShapeDtypeStruct((D_, BC_, T_), _F32),
        compiler_params=pltpu.CompilerParams(
            dimension_semantics=("parallel", "arbitrary")),
        name="s4_inproj",
    )(x, W_in, b_in2)

    u3 = u
    yconv = pl.pallas_call(
        _conv_kernel,
        grid=(D_ // DTC,),
        in_specs=[
            pl.BlockSpec((DTC, BC_, T_), lambda dj: (dj, 0, 0)),
            pl.BlockSpec((DTC, N_), lambda dj: (dj, 0)),
            pl.BlockSpec((DTC, N_), lambda dj: (dj, 0)),
            pl.BlockSpec((DTC, N_), lambda dj: (dj, 0)),
            pl.BlockSpec((DTC, 1), lambda dj: (dj, 0)),
            pl.BlockSpec((DTC, 1), lambda dj: (dj, 0)),
        ],
        out_specs=pl.BlockSpec((DTC, BC_, T_), lambda dj: (dj, 0, 0)),
        out_shape=jax.ShapeDtypeStruct((D_, BC_, T_), jnp.bfloat16),
        compiler_params=pltpu.CompilerParams(
            dimension_semantics=("parallel",)),
        name="s4_ssm_conv",
    )(u3, A_log, B, C, dt2, dd2)

    y2 = yconv.reshape(D_, B_ * L_)
    out = pl.pallas_call(
        _outproj_kernel,
        grid=(D_ // DTO, B_),
        in_specs=[
            pl.BlockSpec((D_, L_), lambda do, b: (0, b)),
            pl.BlockSpec((DTO, D_), lambda do, b: (do, 0)),
            pl.BlockSpec((1, DTO), lambda do, b: (0, do)),
        ],
        out_specs=pl.BlockSpec((1, L_, DTO), lambda do, b: (b, 0, do)),
        out_shape=jax.ShapeDtypeStruct((B_, L_, D_), _F32),
        compiler_params=pltpu.CompilerParams(
            dimension_semantics=("parallel", "arbitrary")),
        name="s4_outproj",
    )(y2, W_out, b_out2)
    return out
